# baseline (speedup 1.0000x reference)
import functools
import math

import jax
import jax.numpy as jnp
from jax import lax
from jax.experimental import pallas as pl
from jax.experimental.pallas import tpu as pltpu
from jax.experimental.pallas import tpu_sc as plsc

N = 10000
E = 320000
G = 256
S = 10
H = 4
HID = 64
NW = 32
NSUB = 16
EPW = E // NW
CH = 80
NCH = EPW // CH
NP = 10240
RPS = NP // NSUB
ZR = 128


def _sc_mesh():
    return plsc.VectorSubcoreMesh(core_axis_name="c", subcore_axis_name="s")


_SC_PARAMS = pltpu.CompilerParams(use_tc_tiling_on_sc=False)


def _zero_vmem(buf, rows, f):
    def body(r, _):
        for j in range(f // 16):
            buf[r, pl.ds(j * 16, 16)] = jnp.zeros((16,), jnp.float32)
        return 0
    lax.fori_loop(0, rows, body, 0)


def _sc_degree(col3):
    @functools.partial(
        pl.kernel,
        mesh=_sc_mesh(),
        out_type=jax.ShapeDtypeStruct((2 * NP, 16), jnp.float32),
        compiler_params=_SC_PARAMS,
        scratch_types=[
            pltpu.VMEM((NCH, CH), jnp.int32),
            pltpu.VMEM((CH, 16), jnp.float32),
            pltpu.VMEM((ZR, 16), jnp.float32),
            pltpu.VMEM_SHARED((NP, 16), jnp.float32),
        ],
    )
    def k(col_hbm, out_hbm, col_v, ones_v, zbuf, shared):
        c = lax.axis_index("c")
        s = lax.axis_index("s")
        wid = s * 2 + c
        _zero_vmem(zbuf, ZR, 16)

        def fill_ones(r, _):
            ones_v[r, pl.ds(0, 16)] = jnp.full((16,), 1.0, jnp.float32)
            return 0
        lax.fori_loop(0, CH, fill_ones, 0)

        for j in range(RPS // ZR):
            pltpu.sync_copy(zbuf, shared.at[pl.ds(s * RPS + j * ZR, ZR)])
        plsc.subcore_barrier()

        pltpu.sync_copy(col_hbm.at[wid], col_v)

        def body(i, _):
            pltpu.sync_copy(ones_v, shared.at[col_v.at[i]], add=True)
            return 0
        lax.fori_loop(0, NCH, body, 0)

        plsc.subcore_barrier()
        pltpu.sync_copy(shared.at[pl.ds(s * RPS, RPS)],
                        out_hbm.at[pl.ds(c * NP + s * RPS, RPS)])

    return k(col3)


def _sc_aggregate(y, row3, col3, f, stage=False):
    scratch = [
        pltpu.VMEM((NCH, CH), jnp.int32),
        pltpu.VMEM((NCH, CH), jnp.int32),
        pltpu.VMEM((CH, f), jnp.float32),
        pltpu.VMEM((CH, f), jnp.float32),
        pltpu.VMEM_SHARED((NP, f), jnp.float32),
        pltpu.SemaphoreType.DMA,
        pltpu.SemaphoreType.DMA,
        pltpu.SemaphoreType.DMA,
        pltpu.SemaphoreType.DMA,
    ]
    if stage:
        scratch.append(pltpu.VMEM_SHARED((N, f), jnp.float32))

    @functools.partial(
        pl.kernel,
        mesh=_sc_mesh(),
        out_type=jax.ShapeDtypeStruct((2 * NP, f), jnp.float32),
        compiler_params=_SC_PARAMS,
        scratch_types=scratch,
    )
    def k(y_hbm, row_hbm, col_hbm, out_hbm, row_v, col_v, ga, gb,
          shared, sema, semb, ssa, ssb, *sy_rest):
        c = lax.axis_index("c")
        s = lax.axis_index("s")
        wid = s * 2 + c
        _zero_vmem(ga, CH, f)
        for j in range(RPS // CH):
            pltpu.sync_copy(ga, shared.at[pl.ds(s * RPS + j * CH, CH)])
        if stage:
            src = sy_rest[0]
            chunk = 640
            @pl.when(s < NSUB - 1)
            def _():
                pltpu.sync_copy(y_hbm.at[pl.ds(s * chunk, chunk)],
                                src.at[pl.ds(s * chunk, chunk)])
            @pl.when(s == NSUB - 1)
            def _():
                rem = N - (NSUB - 1) * chunk
                pltpu.sync_copy(y_hbm.at[pl.ds((NSUB - 1) * chunk, rem)],
                                src.at[pl.ds((NSUB - 1) * chunk, rem)])
        else:
            src = y_hbm
        plsc.subcore_barrier()

        pltpu.sync_copy(row_hbm.at[wid], row_v)
        pltpu.sync_copy(col_hbm.at[wid], col_v)

        def fire_g(i, buf, sem):
            pltpu.make_async_copy(src.at[row_v.at[i]], buf, sem).start()

        def wait_g(i, buf, sem):
            pltpu.make_async_copy(src.at[row_v.at[i]], buf, sem).wait()

        def fire_s(i, buf, sem):
            pltpu.make_async_copy(buf, shared.at[col_v.at[i]], sem).start(
                add=True)

        def wait_s(i, buf, sem):
            pltpu.make_async_copy(buf, shared.at[col_v.at[i]], sem).wait()

        fire_g(0, ga, sema)
        fire_g(1, gb, semb)

        def body(k2, _):
            i0 = k2 * 2
            wait_g(i0, ga, sema)
            fire_s(i0, ga, ssa)
            wait_g(i0 + 1, gb, semb)
            fire_s(i0 + 1, gb, ssb)
            wait_s(i0, ga, ssa)
            fire_g(i0 + 2, ga, sema)
            wait_s(i0 + 1, gb, ssb)
            fire_g(i0 + 3, gb, semb)
            return 0
        lax.fori_loop(0, (NCH - 3) // 2, body, 0)
        wait_g(NCH - 3, ga, sema)
        fire_s(NCH - 3, ga, ssa)
        wait_g(NCH - 2, gb, semb)
        fire_s(NCH - 2, gb, ssb)
        wait_s(NCH - 3, ga, ssa)
        fire_g(NCH - 1, ga, sema)
        wait_s(NCH - 2, gb, ssb)
        wait_g(NCH - 1, ga, sema)
        fire_s(NCH - 1, ga, ssa)
        wait_s(NCH - 1, ga, ssa)

        plsc.subcore_barrier()
        pltpu.sync_copy(shared.at[pl.ds(s * RPS, RPS)],
                        out_hbm.at[pl.ds(c * NP + s * RPS, RPS)])

    return k(y, row3, col3)


def _dinv_of(deg2):
    d = deg2[0, :, 0:1] + deg2[1, :, 0:1] + 1.0
    return lax.rsqrt(d)


def _tc_first(x, w1, deg2):
    def body(x_ref, w_ref, deg_ref, y_ref):
        dinv = _dinv_of(deg_ref[...])
        y_ref[...] = jnp.dot(x_ref[...], w_ref[...],
                             preferred_element_type=jnp.float32) * dinv
    return pl.pallas_call(
        body,
        out_shape=jax.ShapeDtypeStruct((N, 32), jnp.float32),
    )(x, w1, deg2)


def _tc_mid(agg, y_prev, deg2, b_prev, w_next):
    def body(a_ref, y_ref, deg_ref, b_ref, w_ref, xk_ref, yn_ref):
        dinv = _dinv_of(deg_ref[...])
        a = a_ref[...]
        xk = jax.nn.relu(dinv * (a[0] + a[1] + y_ref[...]) + b_ref[...])
        xk_ref[...] = xk
        yn_ref[...] = jnp.dot(xk, w_ref[...],
                              preferred_element_type=jnp.float32) * dinv
    return pl.pallas_call(
        body,
        out_shape=[jax.ShapeDtypeStruct((N, 32), jnp.float32),
                   jax.ShapeDtypeStruct((N, 32), jnp.float32)],
    )(agg, y_prev, deg2, b_prev, w_next)


def _tc_kv(agg3, y3, deg2, b3, x1, x2, wp1, bp1, wk, wv):
    def body(a_ref, y_ref, deg_ref, b_ref, x1_ref, x2_ref, wp_ref, bp_ref,
             wk_ref, wv_ref, ykv_ref):
        dinv = _dinv_of(deg_ref[...])
        a = a_ref[...]
        x3 = jax.nn.relu(dinv * (a[0] + a[1] + y_ref[...]) + b_ref[...])
        wp = wp_ref[...]
        xh = (jnp.dot(x1_ref[...], wp[0:32],
                      preferred_element_type=jnp.float32)
              + jnp.dot(x2_ref[...], wp[32:64],
                        preferred_element_type=jnp.float32)
              + jnp.dot(x3, wp[64:96], preferred_element_type=jnp.float32)
              + bp_ref[...])
        yk = jnp.dot(xh, wk_ref[...], preferred_element_type=jnp.float32) * dinv
        yv = jnp.dot(xh, wv_ref[...], preferred_element_type=jnp.float32) * dinv
        ykv_ref[...] = jnp.concatenate([yk, yv], axis=1)
    return pl.pallas_call(
        body,
        out_shape=jax.ShapeDtypeStruct((N, 2 * HID), jnp.float32),
    )(agg3, y3, deg2, b3, x1, x2, wp1, bp1, wk, wv)


def _tc_pool(aggkv, ykv, deg2, batch3, bk, bv, sg0, wq, bq):
    blk = 1000
    nblk = N // blk

    def body(a_ref, ykv_ref, deg_ref, b_ref, bk_ref, bv_ref, sg_ref, wq_ref,
             bq_ref, out_ref):
        i = pl.program_id(0)
        dinv = _dinv_of(deg_ref[...])
        a = a_ref[...]
        ykvb = ykv_ref[...]
        kn = dinv * (a[0, :, 0:HID] + a[1, :, 0:HID] + ykvb[:, 0:HID]) + bk_ref[...]
        vn = dinv * (a[0, :, HID:] + a[1, :, HID:] + ykvb[:, HID:]) + bv_ref[...]
        qp0 = jnp.dot(sg_ref[...], wq_ref[...],
                      preferred_element_type=jnp.float32) + bq_ref[...]
        inv_sqrt = 1.0 / math.sqrt(HID)
        parts = []
        for h in range(H):
            sc = lax.dot_general(
                kn[:, h * 16:(h + 1) * 16], qp0[:, h * 16:(h + 1) * 16],
                (((1,), (1,)), ((), ())),
                preferred_element_type=jnp.float32) * inv_sqrt
            m = jnp.max(sc, axis=1, keepdims=True)
            e = jnp.exp(sc - m)
            parts.append(e / jnp.sum(e, axis=1, keepdims=True))
        a40 = jnp.concatenate(parts, axis=1)
        ji = lax.broadcasted_iota(jnp.int32, (H * S, S * HID), 0)
        ci = lax.broadcasted_iota(jnp.int32, (H * S, S * HID), 1)
        em = jnp.logical_and(ci // HID == ji % S,
                             (ci % HID) // 16 == ji // S).astype(jnp.float32)
        pexp = jnp.dot(a40, em, preferred_element_type=jnp.float32)
        vt = jnp.concatenate([vn] * S, axis=1)
        p = pexp * vt
        bb = b_ref[0, 0, :]
        msk = (lax.broadcasted_iota(jnp.int32, (G, blk), 0)
               == bb[None, :]).astype(jnp.float32)
        contrib = jnp.dot(msk, p, preferred_element_type=jnp.float32)

        @pl.when(i == 0)
        def _():
            out_ref[...] = jnp.zeros_like(out_ref)
        out_ref[...] += contrib

    return pl.pallas_call(
        body,
        grid=(nblk,),
        in_specs=[
            pl.BlockSpec((2, blk, 2 * HID), lambda i: (0, i, 0)),
            pl.BlockSpec((blk, 2 * HID), lambda i: (i, 0)),
            pl.BlockSpec((2, blk, 16), lambda i: (0, i, 0)),
            pl.BlockSpec((1, 1, blk), lambda i: (i, 0, 0)),
            pl.BlockSpec((1, HID), lambda i: (0, 0)),
            pl.BlockSpec((1, HID), lambda i: (0, 0)),
            pl.BlockSpec((S, HID), lambda i: (0, 0)),
            pl.BlockSpec((HID, HID), lambda i: (0, 0)),
            pl.BlockSpec((1, HID), lambda i: (0, 0)),
        ],
        out_specs=pl.BlockSpec((G, S * HID), lambda i: (0, 0)),
        out_shape=jax.ShapeDtypeStruct((G, S * HID), jnp.float32),
    )(aggkv, ykv, deg2, batch3, bk, bv, sg0, wq, bq)


def _tc_head(oct_, sg0, p):
    pg, ps, pi = p["pma_g"], p["sab"], p["pma_i"]
    eye = jnp.eye(S, dtype=jnp.float32)

    def big(lin):
        return jnp.kron(eye, lin["W"].T)

    def bigb(lin):
        return jnp.broadcast_to(jnp.tile(lin["b"], S)[:, None], (S * HID, G))

    def colb(lin):
        return jnp.broadcast_to(lin["b"][:, None], (lin["b"].shape[0], G))

    args = [
        oct_, sg0, pg["fc_q"]["W"], pg["fc_q"]["b"].reshape(1, HID),
        big(pg["fc_o"]), bigb(pg["fc_o"]),
        big(ps["fc_q"]), bigb(ps["fc_q"]),
        big(ps["layer_k"]), bigb(ps["layer_k"]),
        big(ps["layer_v"]), bigb(ps["layer_v"]),
        big(ps["fc_o"]), bigb(ps["fc_o"]),
        p["pma_i_S"].reshape(1, HID),
        pi["fc_q"]["W"].T, colb(pi["fc_q"]),
        big(pi["layer_v"]), bigb(pi["layer_v"]),
        pi["fc_o"]["W"].T, colb(pi["fc_o"]),
        p["pool_lin2"]["W"].T, colb(p["pool_lin2"]),
        p["lin1"]["W"].T, colb(p["lin1"]),
        p["lin2"]["W"].T, colb(p["lin2"]),
    ]

    def mm(a, b):
        return jnp.dot(a, b, preferred_element_type=jnp.float32)

    def body(oc_ref, sg_ref, wqg, bqg, wogb, bogc, wqsb, bqsc, wksb, bksc,
             wvsb, bvsc, wosb, bosc, si, wqit, bqic, wvib, bvic, woit, boic,
             wp2t, bp2c, wl1t, bl1c, wl2t, bl2c, out_ref):
        f32 = jnp.float32
        SH = S * HID
        qp0 = mm(sg_ref[...], wqg[...]) + bqg[...]
        sel_s = (lax.broadcasted_iota(jnp.int32, (SH, S), 0) // HID
                 == lax.broadcasted_iota(jnp.int32, (SH, S), 1)).astype(f32)
        sel_d = (lax.broadcasted_iota(jnp.int32, (SH, HID), 0) % HID
                 == lax.broadcasted_iota(jnp.int32, (SH, HID), 1)).astype(f32)
        qp0col = mm(mm(sel_s, qp0) * sel_d, jnp.ones((HID, 1), f32))
        qp0mat = mm(qp0col, jnp.ones((1, G), f32))

        bx = qp0mat + oc_ref[...]
        o1 = bx + jax.nn.relu(mm(wogb[...], bx) + bogc[...])

        qs = mm(wqsb[...], o1) + bqsc[...]
        ks = mm(wksb[...], o1) + bksc[...]
        vs = mm(wvsb[...], o1) + bvsc[...]
        ji = lax.broadcasted_iota(jnp.int32, (S * H, SH), 0)
        ci = lax.broadcasted_iota(jnp.int32, (S * H, SH), 1)
        hs = jnp.logical_and(ci // HID == ji // H,
                             (ci % HID) // 16 == ji % H).astype(f32)
        ji2 = lax.broadcasted_iota(jnp.int32, (SH, S * H), 1)
        ci2 = lax.broadcasted_iota(jnp.int32, (SH, S * H), 0)
        hexp = jnp.logical_and(ci2 // HID == ji2 // H,
                               (ci2 % HID) // 16 == ji2 % H).astype(f32)
        inv_sqrt = 1.0 / math.sqrt(HID)
        acc = jnp.zeros((SH, G), f32)
        for t in range(S):
            kblk = ks[t * HID:(t + 1) * HID]
            ktile = jnp.concatenate([kblk] * S, axis=0)
            sc_t = mm(hs, qs * ktile) * inv_sqrt
            scr = sc_t.reshape(S, H, G)
            m = jnp.max(scr, axis=0)
            e = jnp.exp(scr - m[None])
            a_t = (e / jnp.sum(e, axis=0)[None]).reshape(S * H, G)
            vblk = vs[t * HID:(t + 1) * HID]
            vtile = jnp.concatenate([vblk] * S, axis=0)
            acc = acc + mm(hexp, a_t) * vtile
        o2 = qs + acc
        o2 = o2 + jax.nn.relu(mm(wosb[...], o2) + bosc[...])

        vi = mm(wvib[...], o2) + bvic[...]
        sumv = jnp.sum(vi.reshape(S, HID, G), axis=0)
        qpic = lax.dot_general(wqit[...], si[...],
                               (((1,), (1,)), ((), ())),
                               preferred_element_type=f32)
        o3 = mm(qpic, jnp.ones((1, G), f32)) + bqic[...] + sumv
        o3 = o3 + jax.nn.relu(mm(woit[...], o3) + boic[...])

        t1 = mm(wp2t[...], o3) + bp2c[...]
        t2 = jax.nn.relu(mm(wl1t[...], t1) + bl1c[...])
        lg = mm(wl2t[...], t2) + bl2c[...]
        mx = jnp.max(lg, axis=0, keepdims=True)
        lse = mx + jnp.log(jnp.sum(jnp.exp(lg - mx), axis=0, keepdims=True))
        out_ref[...] = lg - lse

    return pl.pallas_call(
        body,
        out_shape=jax.ShapeDtypeStruct((2, G), jnp.float32),
    )(*args)


def kernel(x, edge_index, batch, params):
    row3 = edge_index[0].astype(jnp.int32).reshape(NW, NCH, CH)
    col3 = edge_index[1].astype(jnp.int32).reshape(NW, NCH, CH)
    batch3 = batch.astype(jnp.int32).reshape(N // 1000, 1, 1000)

    deg_flat = _sc_degree(col3)
    deg2 = deg_flat.reshape(2, NP, 16)[:, :N]

    y1 = _tc_first(x, params["conv1"]["W"], deg2)
    agg1 = _sc_aggregate(y1, row3, col3, 32, stage=True).reshape(2, NP, 32)[:, :N]
    x1, y2 = _tc_mid(agg1, y1, deg2, params["conv1"]["b"].reshape(1, -1),
                     params["conv2"]["W"])
    agg2 = _sc_aggregate(y2, row3, col3, 32, stage=True).reshape(2, NP, 32)[:, :N]
    x2, y3 = _tc_mid(agg2, y2, deg2, params["conv2"]["b"].reshape(1, -1),
                     params["conv3"]["W"])
    agg3 = _sc_aggregate(y3, row3, col3, 32, stage=True).reshape(2, NP, 32)[:, :N]

    pg = params["pma_g"]
    ykv = _tc_kv(agg3, y3, deg2, params["conv3"]["b"].reshape(1, -1), x1, x2,
                 params["pool_lin1"]["W"],
                 params["pool_lin1"]["b"].reshape(1, -1),
                 pg["layer_k"]["W"], pg["layer_v"]["W"])
    aggkv = _sc_aggregate(ykv, row3, col3, 2 * HID).reshape(2, NP, 2 * HID)[:, :N]

    oc = _tc_pool(aggkv, ykv, deg2, batch3,
                  pg["layer_k"]["b"].reshape(1, -1),
                  pg["layer_v"]["b"].reshape(1, -1),
                  params["pma_g_S"].reshape(S, HID),
                  pg["fc_q"]["W"], pg["fc_q"]["b"].reshape(1, -1))
    oct_ = oc.T
    return _tc_head(oct_, params["pma_g_S"].reshape(S, HID), params).T


# speedup vs baseline: 1.1046x; 1.1046x over previous
import functools
import math

import jax
import jax.numpy as jnp
from jax import lax
from jax.experimental import pallas as pl
from jax.experimental.pallas import tpu as pltpu
from jax.experimental.pallas import tpu_sc as plsc

N = 10000
E = 320000
G = 256
S = 10
H = 4
HID = 64
NW = 32
NSUB = 16
EPW = E // NW
CH = 80
NCH = EPW // CH
NP = 10240
RPS = NP // NSUB
ZR = 128


def _sc_mesh():
    return plsc.VectorSubcoreMesh(core_axis_name="c", subcore_axis_name="s")


_SC_PARAMS = pltpu.CompilerParams(use_tc_tiling_on_sc=False)


def _zero_vmem(buf, rows, f):
    def body(r, _):
        for j in range(f // 16):
            buf[r, pl.ds(j * 16, 16)] = jnp.zeros((16,), jnp.float32)
        return 0
    lax.fori_loop(0, rows, body, 0)


def _sc_degree(col3):
    @functools.partial(
        pl.kernel,
        mesh=_sc_mesh(),
        out_type=jax.ShapeDtypeStruct((2 * NP, 16), jnp.float32),
        compiler_params=_SC_PARAMS,
        scratch_types=[
            pltpu.VMEM((NCH, CH), jnp.int32),
            pltpu.VMEM((CH, 16), jnp.float32),
            pltpu.VMEM((ZR, 16), jnp.float32),
            pltpu.VMEM_SHARED((NP, 16), jnp.float32),
        ],
    )
    def k(col_hbm, out_hbm, col_v, ones_v, zbuf, shared):
        c = lax.axis_index("c")
        s = lax.axis_index("s")
        wid = s * 2 + c
        _zero_vmem(zbuf, ZR, 16)

        def fill_ones(r, _):
            ones_v[r, pl.ds(0, 16)] = jnp.full((16,), 1.0, jnp.float32)
            return 0
        lax.fori_loop(0, CH, fill_ones, 0)

        for j in range(RPS // ZR):
            pltpu.sync_copy(zbuf, shared.at[pl.ds(s * RPS + j * ZR, ZR)])
        plsc.subcore_barrier()

        pltpu.sync_copy(col_hbm.at[wid], col_v)

        def body(i, _):
            pltpu.sync_copy(ones_v, shared.at[col_v.at[i]], add=True)
            return 0
        lax.fori_loop(0, NCH, body, 0)

        plsc.subcore_barrier()
        pltpu.sync_copy(shared.at[pl.ds(s * RPS, RPS)],
                        out_hbm.at[pl.ds(c * NP + s * RPS, RPS)])

    return k(col3)


def _sc_aggregate(y, row3, col3, f, stage=False):
    scratch = [
        pltpu.VMEM((NCH, CH), jnp.int32),
        pltpu.VMEM((NCH, CH), jnp.int32),
        pltpu.VMEM((CH, f), jnp.float32),
        pltpu.VMEM((CH, f), jnp.float32),
        pltpu.VMEM_SHARED((NP, f), jnp.float32),
        pltpu.SemaphoreType.DMA,
        pltpu.SemaphoreType.DMA,
    ]
    if stage:
        scratch.append(pltpu.VMEM_SHARED((N, f), jnp.float32))

    @functools.partial(
        pl.kernel,
        mesh=_sc_mesh(),
        out_type=jax.ShapeDtypeStruct((2 * NP, f), jnp.float32),
        compiler_params=_SC_PARAMS,
        scratch_types=scratch,
    )
    def k(y_hbm, row_hbm, col_hbm, out_hbm, row_v, col_v, ga, gb,
          shared, sema, semb, *sy_rest):
        c = lax.axis_index("c")
        s = lax.axis_index("s")
        wid = s * 2 + c
        _zero_vmem(ga, CH, f)
        for j in range(RPS // CH):
            pltpu.sync_copy(ga, shared.at[pl.ds(s * RPS + j * CH, CH)])
        if stage:
            src = sy_rest[0]
            chunk = 640
            @pl.when(s < NSUB - 1)
            def _():
                pltpu.sync_copy(y_hbm.at[pl.ds(s * chunk, chunk)],
                                src.at[pl.ds(s * chunk, chunk)])
            @pl.when(s == NSUB - 1)
            def _():
                rem = N - (NSUB - 1) * chunk
                pltpu.sync_copy(y_hbm.at[pl.ds((NSUB - 1) * chunk, rem)],
                                src.at[pl.ds((NSUB - 1) * chunk, rem)])
        else:
            src = y_hbm
        plsc.subcore_barrier()

        pltpu.sync_copy(row_hbm.at[wid], row_v)
        pltpu.sync_copy(col_hbm.at[wid], col_v)

        def fire(i, buf, sem):
            pltpu.make_async_copy(src.at[row_v.at[i]], buf, sem).start()

        def drain(i, buf, sem):
            pltpu.make_async_copy(src.at[row_v.at[i]], buf, sem).wait()
            pltpu.sync_copy(buf, shared.at[col_v.at[i]], add=True)

        fire(0, ga, sema)

        def body(k2, _):
            i0 = k2 * 2
            fire(i0 + 1, gb, semb)
            drain(i0, ga, sema)
            fire(i0 + 2, ga, sema)
            drain(i0 + 1, gb, semb)
            return 0
        lax.fori_loop(0, (NCH - 1) // 2, body, 0)
        drain(NCH - 1, ga, sema)

        plsc.subcore_barrier()
        pltpu.sync_copy(shared.at[pl.ds(s * RPS, RPS)],
                        out_hbm.at[pl.ds(c * NP + s * RPS, RPS)])

    return k(y, row3, col3)


def _dinv_of(deg2):
    d = deg2[0, :, 0:1] + deg2[1, :, 0:1] + 1.0
    return lax.rsqrt(d)


def _tc_first(x, w1, deg2):
    def body(x_ref, w_ref, deg_ref, y_ref):
        dinv = _dinv_of(deg_ref[...])
        y_ref[...] = jnp.dot(x_ref[...], w_ref[...],
                             preferred_element_type=jnp.float32) * dinv
    return pl.pallas_call(
        body,
        out_shape=jax.ShapeDtypeStruct((N, 32), jnp.float32),
    )(x, w1, deg2)


def _tc_mid(agg, y_prev, deg2, b_prev, w_next):
    def body(a_ref, y_ref, deg_ref, b_ref, w_ref, xk_ref, yn_ref):
        dinv = _dinv_of(deg_ref[...])
        a = a_ref[...]
        xk = jax.nn.relu(dinv * (a[0] + a[1] + y_ref[...]) + b_ref[...])
        xk_ref[...] = xk
        yn_ref[...] = jnp.dot(xk, w_ref[...],
                              preferred_element_type=jnp.float32) * dinv
    return pl.pallas_call(
        body,
        out_shape=[jax.ShapeDtypeStruct((N, 32), jnp.float32),
                   jax.ShapeDtypeStruct((N, 32), jnp.float32)],
    )(agg, y_prev, deg2, b_prev, w_next)


def _tc_kv(agg3, y3, deg2, b3, x1, x2, wp1, bp1, wk, wv):
    def body(a_ref, y_ref, deg_ref, b_ref, x1_ref, x2_ref, wp_ref, bp_ref,
             wk_ref, wv_ref, ykv_ref):
        dinv = _dinv_of(deg_ref[...])
        a = a_ref[...]
        x3 = jax.nn.relu(dinv * (a[0] + a[1] + y_ref[...]) + b_ref[...])
        wp = wp_ref[...]
        xh = (jnp.dot(x1_ref[...], wp[0:32],
                      preferred_element_type=jnp.float32)
              + jnp.dot(x2_ref[...], wp[32:64],
                        preferred_element_type=jnp.float32)
              + jnp.dot(x3, wp[64:96], preferred_element_type=jnp.float32)
              + bp_ref[...])
        yk = jnp.dot(xh, wk_ref[...], preferred_element_type=jnp.float32) * dinv
        yv = jnp.dot(xh, wv_ref[...], preferred_element_type=jnp.float32) * dinv
        ykv_ref[...] = jnp.concatenate([yk, yv], axis=1)
    return pl.pallas_call(
        body,
        out_shape=jax.ShapeDtypeStruct((N, 2 * HID), jnp.float32),
    )(agg3, y3, deg2, b3, x1, x2, wp1, bp1, wk, wv)


def _tc_pool(aggkv, ykv, deg2, batch3, bk, bv, sg0, wq, bq):
    blk = 1000
    nblk = N // blk

    def body(a_ref, ykv_ref, deg_ref, b_ref, bk_ref, bv_ref, sg_ref, wq_ref,
             bq_ref, out_ref):
        i = pl.program_id(0)
        dinv = _dinv_of(deg_ref[...])
        a = a_ref[...]
        ykvb = ykv_ref[...]
        kn = dinv * (a[0, :, 0:HID] + a[1, :, 0:HID] + ykvb[:, 0:HID]) + bk_ref[...]
        vn = dinv * (a[0, :, HID:] + a[1, :, HID:] + ykvb[:, HID:]) + bv_ref[...]
        qp0 = jnp.dot(sg_ref[...], wq_ref[...],
                      preferred_element_type=jnp.float32) + bq_ref[...]
        inv_sqrt = 1.0 / math.sqrt(HID)
        parts = []
        for h in range(H):
            sc = lax.dot_general(
                kn[:, h * 16:(h + 1) * 16], qp0[:, h * 16:(h + 1) * 16],
                (((1,), (1,)), ((), ())),
                preferred_element_type=jnp.float32) * inv_sqrt
            m = jnp.max(sc, axis=1, keepdims=True)
            e = jnp.exp(sc - m)
            parts.append(e / jnp.sum(e, axis=1, keepdims=True))
        a40 = jnp.concatenate(parts, axis=1)
        ji = lax.broadcasted_iota(jnp.int32, (H * S, S * HID), 0)
        ci = lax.broadcasted_iota(jnp.int32, (H * S, S * HID), 1)
        em = jnp.logical_and(ci // HID == ji % S,
                             (ci % HID) // 16 == ji // S).astype(jnp.float32)
        pexp = jnp.dot(a40, em, preferred_element_type=jnp.float32)
        vt = jnp.concatenate([vn] * S, axis=1)
        p = pexp * vt
        bb = b_ref[0, 0, :]
        msk = (lax.broadcasted_iota(jnp.int32, (G, blk), 0)
               == bb[None, :]).astype(jnp.float32)
        contrib = jnp.dot(msk, p, preferred_element_type=jnp.float32)

        @pl.when(i == 0)
        def _():
            out_ref[...] = jnp.zeros_like(out_ref)
        out_ref[...] += contrib

    return pl.pallas_call(
        body,
        grid=(nblk,),
        in_specs=[
            pl.BlockSpec((2, blk, 2 * HID), lambda i: (0, i, 0)),
            pl.BlockSpec((blk, 2 * HID), lambda i: (i, 0)),
            pl.BlockSpec((2, blk, 16), lambda i: (0, i, 0)),
            pl.BlockSpec((1, 1, blk), lambda i: (i, 0, 0)),
            pl.BlockSpec((1, HID), lambda i: (0, 0)),
            pl.BlockSpec((1, HID), lambda i: (0, 0)),
            pl.BlockSpec((S, HID), lambda i: (0, 0)),
            pl.BlockSpec((HID, HID), lambda i: (0, 0)),
            pl.BlockSpec((1, HID), lambda i: (0, 0)),
        ],
        out_specs=pl.BlockSpec((G, S * HID), lambda i: (0, 0)),
        out_shape=jax.ShapeDtypeStruct((G, S * HID), jnp.float32),
    )(aggkv, ykv, deg2, batch3, bk, bv, sg0, wq, bq)


def _tc_head(oct_, sg0, p):
    pg, ps, pi = p["pma_g"], p["sab"], p["pma_i"]
    eye = jnp.eye(S, dtype=jnp.float32)

    def big(lin):
        return jnp.kron(eye, lin["W"].T)

    def bigb(lin):
        return jnp.broadcast_to(jnp.tile(lin["b"], S)[:, None], (S * HID, G))

    def colb(lin):
        return jnp.broadcast_to(lin["b"][:, None], (lin["b"].shape[0], G))

    args = [
        oct_, sg0, pg["fc_q"]["W"], pg["fc_q"]["b"].reshape(1, HID),
        big(pg["fc_o"]), bigb(pg["fc_o"]),
        big(ps["fc_q"]), bigb(ps["fc_q"]),
        big(ps["layer_k"]), bigb(ps["layer_k"]),
        big(ps["layer_v"]), bigb(ps["layer_v"]),
        big(ps["fc_o"]), bigb(ps["fc_o"]),
        p["pma_i_S"].reshape(1, HID),
        pi["fc_q"]["W"].T, colb(pi["fc_q"]),
        big(pi["layer_v"]), bigb(pi["layer_v"]),
        pi["fc_o"]["W"].T, colb(pi["fc_o"]),
        p["pool_lin2"]["W"].T, colb(p["pool_lin2"]),
        p["lin1"]["W"].T, colb(p["lin1"]),
        p["lin2"]["W"].T, colb(p["lin2"]),
    ]

    def mm(a, b):
        return jnp.dot(a, b, preferred_element_type=jnp.float32)

    def body(oc_ref, sg_ref, wqg, bqg, wogb, bogc, wqsb, bqsc, wksb, bksc,
             wvsb, bvsc, wosb, bosc, si, wqit, bqic, wvib, bvic, woit, boic,
             wp2t, bp2c, wl1t, bl1c, wl2t, bl2c, out_ref):
        f32 = jnp.float32
        SH = S * HID
        qp0 = mm(sg_ref[...], wqg[...]) + bqg[...]
        sel_s = (lax.broadcasted_iota(jnp.int32, (SH, S), 0) // HID
                 == lax.broadcasted_iota(jnp.int32, (SH, S), 1)).astype(f32)
        sel_d = (lax.broadcasted_iota(jnp.int32, (SH, HID), 0) % HID
                 == lax.broadcasted_iota(jnp.int32, (SH, HID), 1)).astype(f32)
        qp0col = mm(mm(sel_s, qp0) * sel_d, jnp.ones((HID, 1), f32))
        qp0mat = mm(qp0col, jnp.ones((1, G), f32))

        bx = qp0mat + oc_ref[...]
        o1 = bx + jax.nn.relu(mm(wogb[...], bx) + bogc[...])

        qs = mm(wqsb[...], o1) + bqsc[...]
        ks = mm(wksb[...], o1) + bksc[...]
        vs = mm(wvsb[...], o1) + bvsc[...]
        ji = lax.broadcasted_iota(jnp.int32, (S * H, SH), 0)
        ci = lax.broadcasted_iota(jnp.int32, (S * H, SH), 1)
        hs = jnp.logical_and(ci // HID == ji // H,
                             (ci % HID) // 16 == ji % H).astype(f32)
        ji2 = lax.broadcasted_iota(jnp.int32, (SH, S * H), 1)
        ci2 = lax.broadcasted_iota(jnp.int32, (SH, S * H), 0)
        hexp = jnp.logical_and(ci2 // HID == ji2 // H,
                               (ci2 % HID) // 16 == ji2 % H).astype(f32)
        inv_sqrt = 1.0 / math.sqrt(HID)
        acc = jnp.zeros((SH, G), f32)
        for t in range(S):
            kblk = ks[t * HID:(t + 1) * HID]
            ktile = jnp.concatenate([kblk] * S, axis=0)
            sc_t = mm(hs, qs * ktile) * inv_sqrt
            scr = sc_t.reshape(S, H, G)
            m = jnp.max(scr, axis=0)
            e = jnp.exp(scr - m[None])
            a_t = (e / jnp.sum(e, axis=0)[None]).reshape(S * H, G)
            vblk = vs[t * HID:(t + 1) * HID]
            vtile = jnp.concatenate([vblk] * S, axis=0)
            acc = acc + mm(hexp, a_t) * vtile
        o2 = qs + acc
        o2 = o2 + jax.nn.relu(mm(wosb[...], o2) + bosc[...])

        vi = mm(wvib[...], o2) + bvic[...]
        sumv = jnp.sum(vi.reshape(S, HID, G), axis=0)
        qpic = lax.dot_general(wqit[...], si[...],
                               (((1,), (1,)), ((), ())),
                               preferred_element_type=f32)
        o3 = mm(qpic, jnp.ones((1, G), f32)) + bqic[...] + sumv
        o3 = o3 + jax.nn.relu(mm(woit[...], o3) + boic[...])

        t1 = mm(wp2t[...], o3) + bp2c[...]
        t2 = jax.nn.relu(mm(wl1t[...], t1) + bl1c[...])
        lg = mm(wl2t[...], t2) + bl2c[...]
        mx = jnp.max(lg, axis=0, keepdims=True)
        lse = mx + jnp.log(jnp.sum(jnp.exp(lg - mx), axis=0, keepdims=True))
        out_ref[...] = lg - lse

    return pl.pallas_call(
        body,
        out_shape=jax.ShapeDtypeStruct((2, G), jnp.float32),
    )(*args)


def kernel(x, edge_index, batch, params):
    row3 = edge_index[0].astype(jnp.int32).reshape(NW, NCH, CH)
    col3 = edge_index[1].astype(jnp.int32).reshape(NW, NCH, CH)
    batch3 = batch.astype(jnp.int32).reshape(N // 1000, 1, 1000)

    deg_flat = _sc_degree(col3)
    deg2 = deg_flat.reshape(2, NP, 16)[:, :N]

    y1 = _tc_first(x, params["conv1"]["W"], deg2)
    agg1 = _sc_aggregate(y1, row3, col3, 32, stage=True).reshape(2, NP, 32)[:, :N]
    x1, y2 = _tc_mid(agg1, y1, deg2, params["conv1"]["b"].reshape(1, -1),
                     params["conv2"]["W"])
    agg2 = _sc_aggregate(y2, row3, col3, 32, stage=True).reshape(2, NP, 32)[:, :N]
    x2, y3 = _tc_mid(agg2, y2, deg2, params["conv2"]["b"].reshape(1, -1),
                     params["conv3"]["W"])
    agg3 = _sc_aggregate(y3, row3, col3, 32, stage=True).reshape(2, NP, 32)[:, :N]

    pg = params["pma_g"]
    ykv = _tc_kv(agg3, y3, deg2, params["conv3"]["b"].reshape(1, -1), x1, x2,
                 params["pool_lin1"]["W"],
                 params["pool_lin1"]["b"].reshape(1, -1),
                 pg["layer_k"]["W"], pg["layer_v"]["W"])
    aggkv = _sc_aggregate(ykv, row3, col3, 2 * HID).reshape(2, NP, 2 * HID)[:, :N]

    oc = _tc_pool(aggkv, ykv, deg2, batch3,
                  pg["layer_k"]["b"].reshape(1, -1),
                  pg["layer_v"]["b"].reshape(1, -1),
                  params["pma_g_S"].reshape(S, HID),
                  pg["fc_q"]["W"], pg["fc_q"]["b"].reshape(1, -1))
    oct_ = oc.T
    return _tc_head(oct_, params["pma_g_S"].reshape(S, HID), params).T
